# baseline (speedup 1.0000x reference)
import functools

import jax
import jax.numpy as jnp
from jax import lax
from jax.experimental import pallas as pl
from jax.experimental.pallas import tpu as pltpu
from jax.experimental.pallas import tpu_sc as plsc

NUM_EXPERTS = 64
TOP_K = 8
ROWS = 32768
DIM = 4096
NW = 32
LANES = 16

CHUNKS = 8
CROWS = ROWS // CHUNKS
RPW = CROWS // NW
GROUPS = RPW // LANES

BLK = 1024
NBLK = CROWS // BLK
WPB = BLK // RPW


def _matmul_body(x_ref, w_ref, b_ref, out_ref):
    logits = lax.dot_general(
        w_ref[...], x_ref[...],
        dimension_numbers=(((1,), (1,)), ((), ())),
        preferred_element_type=jnp.float32,
    )
    logits = logits + b_ref[...]
    for w in range(WPB):
        out_ref[w] = logits[:, w * RPW:(w + 1) * RPW]


def _logits_chunk(x, W, b2, c):
    return pl.pallas_call(
        _matmul_body,
        grid=(NBLK,),
        in_specs=[
            pl.BlockSpec((BLK, DIM), lambda i, c=c: (c * NBLK + i, 0)),
            pl.BlockSpec((NUM_EXPERTS, DIM), lambda i: (0, 0)),
            pl.BlockSpec((NUM_EXPERTS, 1), lambda i: (0, 0)),
        ],
        out_specs=pl.BlockSpec((WPB, NUM_EXPERTS, RPW), lambda i: (i, 0, 0)),
        out_shape=jax.ShapeDtypeStruct((NW, NUM_EXPERTS, RPW), jnp.float32),
    )(x, W, b2)


def _sc_topk(logits3):
    mesh = plsc.VectorSubcoreMesh(core_axis_name="c", subcore_axis_name="s")

    @functools.partial(
        pl.kernel,
        mesh=mesh,
        out_type=[
            jax.ShapeDtypeStruct((NW, TOP_K, RPW), jnp.int32),
            jax.ShapeDtypeStruct((NW, TOP_K, RPW), jnp.float32),
        ],
        scratch_types=[
            pltpu.VMEM((NUM_EXPERTS, RPW), jnp.float32),
            pltpu.VMEM((TOP_K, RPW), jnp.int32),
            pltpu.VMEM((TOP_K, RPW), jnp.float32),
        ],
    )
    def k(lg_hbm, idx_hbm, w_hbm, slab, idx_v, w_v):
        wid = lax.axis_index("s") * 2 + lax.axis_index("c")
        pltpu.sync_copy(lg_hbm.at[wid], slab)

        def group(g, carry):
            r0 = g * LANES
            m = [jnp.full((LANES,), -jnp.inf, jnp.float32)] * TOP_K
            mi = [jnp.zeros((LANES,), jnp.int32)] * TOP_K
            for e in range(NUM_EXPERTS):
                v = slab[e, pl.ds(r0, LANES)]
                iv = jnp.full((LANES,), e, jnp.int32)
                c = [v > m[j] for j in range(TOP_K)]
                nm = [jnp.where(c[0], v, m[0])]
                ni = [jnp.where(c[0], iv, mi[0])]
                for j in range(1, TOP_K):
                    nm.append(jnp.where(c[j - 1], m[j - 1],
                                        jnp.where(c[j], v, m[j])))
                    ni.append(jnp.where(c[j - 1], mi[j - 1],
                                        jnp.where(c[j], iv, mi[j])))
                m, mi = nm, ni
            ex = [jnp.exp(m[j] - m[0]) for j in range(TOP_K)]
            s = ex[0]
            for j in range(1, TOP_K):
                s = s + ex[j]
            r = 1.0 / s
            for j in range(TOP_K):
                idx_v[j, pl.ds(r0, LANES)] = mi[j]
                w_v[j, pl.ds(r0, LANES)] = ex[j] * r
            return carry

        lax.fori_loop(0, GROUPS, group, 0)
        pltpu.sync_copy(idx_v, idx_hbm.at[wid])
        pltpu.sync_copy(w_v, w_hbm.at[wid])

    return k(logits3)


def kernel(x, W, b):
    b2 = b.reshape(NUM_EXPERTS, 1)
    idxs, ws = [], []
    for c in range(CHUNKS):
        lg = _logits_chunk(x, W, b2, c)
        idx_c, w_c = _sc_topk(lg)
        idxs.append(idx_c.transpose(0, 2, 1).reshape(CROWS, TOP_K))
        ws.append(w_c.transpose(0, 2, 1).reshape(CROWS, TOP_K))
    return jnp.concatenate(idxs, 0), jnp.concatenate(ws, 0)


# speedup vs baseline: 1.1301x; 1.1301x over previous
import functools

import jax
import jax.numpy as jnp
from jax import lax
from jax.experimental import pallas as pl
from jax.experimental.pallas import tpu as pltpu
from jax.experimental.pallas import tpu_sc as plsc

NUM_EXPERTS = 64
TOP_K = 8
ROWS = 32768
DIM = 4096
NW = 32
LANES = 16

BLK = 1024
CHUNK_ROWS = (16384, 8192, 4096, 4096)


def _matmul_body(wpb, x_ref, w_ref, b_ref, out_ref):
    logits = lax.dot_general(
        w_ref[...], x_ref[...],
        dimension_numbers=(((1,), (1,)), ((), ())),
        preferred_element_type=jnp.float32,
    )
    logits = logits + b_ref[...]
    rpw = BLK // wpb
    for w in range(wpb):
        out_ref[w] = logits[:, w * rpw:(w + 1) * rpw]


def _logits_chunk(x, W, b2, row0, crows):
    rpw = crows // NW
    wpb = BLK // rpw
    nblk = crows // BLK
    blk0 = row0 // BLK
    return pl.pallas_call(
        functools.partial(_matmul_body, wpb),
        grid=(nblk,),
        in_specs=[
            pl.BlockSpec((BLK, DIM), lambda i: (blk0 + i, 0)),
            pl.BlockSpec((NUM_EXPERTS, DIM), lambda i: (0, 0)),
            pl.BlockSpec((NUM_EXPERTS, 1), lambda i: (0, 0)),
        ],
        out_specs=pl.BlockSpec((wpb, NUM_EXPERTS, rpw), lambda i: (i, 0, 0)),
        out_shape=jax.ShapeDtypeStruct((NW, NUM_EXPERTS, rpw), jnp.float32),
    )(x, W, b2)


def _sc_topk(logits3, rpw):
    mesh = plsc.VectorSubcoreMesh(core_axis_name="c", subcore_axis_name="s")
    groups = rpw // LANES

    @functools.partial(
        pl.kernel,
        mesh=mesh,
        out_type=[
            jax.ShapeDtypeStruct((NW, TOP_K, rpw), jnp.int32),
            jax.ShapeDtypeStruct((NW, TOP_K, rpw), jnp.float32),
        ],
        scratch_types=[
            pltpu.VMEM((NUM_EXPERTS, rpw), jnp.float32),
            pltpu.VMEM((TOP_K, rpw), jnp.int32),
            pltpu.VMEM((TOP_K, rpw), jnp.float32),
        ],
    )
    def k(lg_hbm, idx_hbm, w_hbm, slab, idx_v, w_v):
        wid = lax.axis_index("s") * 2 + lax.axis_index("c")
        pltpu.sync_copy(lg_hbm.at[wid], slab)

        def group(g, carry):
            r0 = g * LANES
            m = [jnp.full((LANES,), -jnp.inf, jnp.float32)] * TOP_K
            mi = [jnp.zeros((LANES,), jnp.int32)] * TOP_K
            for e in range(NUM_EXPERTS):
                v = slab[e, pl.ds(r0, LANES)]
                iv = jnp.full((LANES,), e, jnp.int32)
                c = [v > m[j] for j in range(TOP_K)]
                nm = [jnp.where(c[0], v, m[0])]
                ni = [jnp.where(c[0], iv, mi[0])]
                for j in range(1, TOP_K):
                    nm.append(jnp.where(c[j - 1], m[j - 1],
                                        jnp.where(c[j], v, m[j])))
                    ni.append(jnp.where(c[j - 1], mi[j - 1],
                                        jnp.where(c[j], iv, mi[j])))
                m, mi = nm, ni
            ex = [jnp.exp(m[j] - m[0]) for j in range(TOP_K)]
            s = ex[0]
            for j in range(1, TOP_K):
                s = s + ex[j]
            r = 1.0 / s
            for j in range(TOP_K):
                idx_v[j, pl.ds(r0, LANES)] = mi[j]
                w_v[j, pl.ds(r0, LANES)] = ex[j] * r
            return carry

        lax.fori_loop(0, groups, group, 0)
        pltpu.sync_copy(idx_v, idx_hbm.at[wid])
        pltpu.sync_copy(w_v, w_hbm.at[wid])

    return k(logits3)


def kernel(x, W, b):
    b2 = b.reshape(NUM_EXPERTS, 1)
    idxs, ws = [], []
    row0 = 0
    for crows in CHUNK_ROWS:
        lg = _logits_chunk(x, W, b2, row0, crows)
        idx_c, w_c = _sc_topk(lg, crows // NW)
        idxs.append(idx_c.transpose(0, 2, 1).reshape(crows, TOP_K))
        ws.append(w_c.transpose(0, 2, 1).reshape(crows, TOP_K))
        row0 += crows
    return jnp.concatenate(idxs, 0), jnp.concatenate(ws, 0)
